# baseline (speedup 1.0000x reference)
import jax
import jax.numpy as jnp
from jax.experimental import pallas as pl
from jax.experimental.pallas import tpu as pltpu

_E = 8
_D = 1024
_F = 4096
_T = 2048
_NBF = 8
_BF = _F // _NBF
_NT = 1
_BT = _T // _NT


def _moe_body(x_ref, wg_ref, bg_ref, w1_ref, b1_ref, w2_ref, b2_ref,
              out_ref, gates_ref, xb_ref):
    e = pl.program_id(1)
    f = pl.program_id(2)

    @pl.when(jnp.logical_and(e == 0, f == 0))
    def _init():
        logits = jax.lax.dot(
            x_ref[...], wg_ref[...],
            precision=jax.lax.Precision.HIGHEST,
            preferred_element_type=jnp.float32) + bg_ref[...]
        m = jnp.max(logits, axis=1, keepdims=True)
        p = jnp.exp(logits - m)
        gates_ref[...] = p / jnp.sum(p, axis=1, keepdims=True)
        xb_ref[...] = x_ref[...].astype(jnp.bfloat16)
        out_ref[...] = jnp.zeros_like(out_ref)

    gates = gates_ref[...]
    col = jax.lax.broadcasted_iota(jnp.int32, gates.shape, 1)
    g = jnp.sum(jnp.where(col == e, gates, 0.0), axis=1, keepdims=True)

    xb = xb_ref[...]
    half = _BF // 2
    part = None
    for s in range(2):
        w1 = w1_ref[0, :, s * half:(s + 1) * half].astype(jnp.bfloat16)
        h = jax.lax.dot(xb, w1, preferred_element_type=jnp.float32)
        h = (jnp.maximum(h + b1_ref[0, :, s * half:(s + 1) * half],
                         0.0) * g).astype(jnp.bfloat16)
        w2 = w2_ref[0, s * half:(s + 1) * half, :].astype(jnp.bfloat16)
        p = jax.lax.dot(h, w2, preferred_element_type=jnp.float32)
        part = p if part is None else part + p

    @pl.when(f == 0)
    def _bias2():
        out_ref[...] += g * b2_ref[0]

    out_ref[...] += part


def kernel(x, Wg, bg, W1, b1, W2, b2):
    bg2 = bg.reshape(1, _E)
    b1r = b1.reshape(_E, 1, _F)
    b2r = b2.reshape(_E, 1, _D)
    grid = (_NT, _E, _NBF)
    return pl.pallas_call(
        _moe_body,
        grid=grid,
        in_specs=[
            pl.BlockSpec((_BT, _D), lambda t, e, f: (t, 0)),
            pl.BlockSpec((_D, _E), lambda t, e, f: (0, 0)),
            pl.BlockSpec((1, _E), lambda t, e, f: (0, 0)),
            pl.BlockSpec((1, _D, _BF), lambda t, e, f: (e, 0, f)),
            pl.BlockSpec((1, 1, _BF), lambda t, e, f: (e, 0, f)),
            pl.BlockSpec((1, _BF, _D), lambda t, e, f: (e, f, 0)),
            pl.BlockSpec((1, 1, _D), lambda t, e, f: (e, 0, 0)),
        ],
        out_specs=pl.BlockSpec((_BT, _D), lambda t, e, f: (t, 0)),
        out_shape=jax.ShapeDtypeStruct((_T, _D), jnp.float32),
        scratch_shapes=[
            pltpu.VMEM((_BT, _E), jnp.float32),
            pltpu.VMEM((_BT, _D), jnp.bfloat16),
        ],
        compiler_params=pltpu.CompilerParams(
            dimension_semantics=("parallel", "arbitrary", "arbitrary"),
        ),
    )(x, Wg, bg2, W1, b1r, W2, b2r)


# speedup vs baseline: 1.1660x; 1.1660x over previous
import jax
import jax.numpy as jnp
from jax.experimental import pallas as pl
from jax.experimental.pallas import tpu as pltpu

_E = 8
_D = 1024
_F = 4096
_T = 2048
_NBF = 8
_BF = _F // _NBF
_NT = 1
_BT = _T // _NT


def _moe_body(x_ref, wg_ref, bg_ref, w1_ref, b1_ref, w2_ref, b2_ref,
              out_ref, gates_ref, xb_ref):
    e = pl.program_id(1)
    f = pl.program_id(2)

    @pl.when(jnp.logical_and(e == 0, f == 0))
    def _init():
        logits = jax.lax.dot(
            x_ref[...], wg_ref[...],
            precision=jax.lax.Precision.DEFAULT,
            preferred_element_type=jnp.float32) + bg_ref[...]
        m = jnp.max(logits, axis=1, keepdims=True)
        p = jnp.exp(logits - m)
        gates_ref[...] = p / jnp.sum(p, axis=1, keepdims=True)
        xb_ref[...] = x_ref[...].astype(jnp.bfloat16)
        out_ref[...] = jnp.zeros_like(out_ref)

    gates = gates_ref[...]
    col = jax.lax.broadcasted_iota(jnp.int32, gates.shape, 1)
    g = jnp.sum(jnp.where(col == e, gates, 0.0), axis=1, keepdims=True)

    xb = xb_ref[...]
    half = _BF // 2
    part = None
    for s in range(2):
        w1 = w1_ref[0, :, s * half:(s + 1) * half].astype(jnp.bfloat16)
        h = jax.lax.dot(xb, w1, preferred_element_type=jnp.float32)
        h = jnp.maximum(h + b1_ref[0, :, s * half:(s + 1) * half],
                        0.0).astype(jnp.bfloat16)
        w2 = w2_ref[0, s * half:(s + 1) * half, :].astype(jnp.bfloat16)
        p = jax.lax.dot(h, w2, preferred_element_type=jnp.float32)
        part = p if part is None else part + p

    part = part + jnp.where(f == 0, 1.0, 0.0) * b2_ref[0]
    out_ref[...] += g * part


def kernel(x, Wg, bg, W1, b1, W2, b2):
    bg2 = bg.reshape(1, _E)
    b1r = b1.reshape(_E, 1, _F)
    b2r = b2.reshape(_E, 1, _D)
    grid = (_NT, _E, _NBF)
    return pl.pallas_call(
        _moe_body,
        grid=grid,
        in_specs=[
            pl.BlockSpec((_BT, _D), lambda t, e, f: (t, 0)),
            pl.BlockSpec((_D, _E), lambda t, e, f: (0, 0)),
            pl.BlockSpec((1, _E), lambda t, e, f: (0, 0)),
            pl.BlockSpec((1, _D, _BF), lambda t, e, f: (e, 0, f)),
            pl.BlockSpec((1, 1, _BF), lambda t, e, f: (e, 0, f)),
            pl.BlockSpec((1, _BF, _D), lambda t, e, f: (e, f, 0)),
            pl.BlockSpec((1, 1, _D), lambda t, e, f: (e, 0, 0)),
        ],
        out_specs=pl.BlockSpec((_BT, _D), lambda t, e, f: (t, 0)),
        out_shape=jax.ShapeDtypeStruct((_T, _D), jnp.float32),
        scratch_shapes=[
            pltpu.VMEM((_BT, _E), jnp.float32),
            pltpu.VMEM((_BT, _D), jnp.bfloat16),
        ],
        compiler_params=pltpu.CompilerParams(
            dimension_semantics=("parallel", "arbitrary", "arbitrary"),
        ),
    )(x, Wg, bg2, W1, b1r, W2, b2r)
